# baseline (speedup 1.0000x reference)
import functools

import numpy as np

import jax
import jax.numpy as jnp
from jax.experimental import pallas as pl
from jax.experimental.pallas import tpu as pltpu

_TB = 1024
_VMEM = 100 * 1024 * 1024


def _np_band_idx1():
    l = np.arange(8).reshape(8, 1, 1, 1, 1, 1, 1)
    wi = np.arange(28).reshape(1, 28, 1, 1, 1, 1, 1)
    rp = np.arange(2).reshape(1, 1, 2, 1, 1, 1, 1)
    wp = np.arange(2).reshape(1, 1, 1, 2, 1, 1, 1)
    phl = np.arange(2).reshape(1, 1, 1, 1, 2, 1, 1)
    co = np.arange(8).reshape(1, 1, 1, 1, 1, 8, 1)
    pw = np.arange(12).reshape(1, 1, 1, 1, 1, 1, 12)
    kh = (l - 2 * phl - rp) % 8
    kw = (wi - 2 * pw - wp) % 32
    idx = co * 256 + kh * 32 + kw
    return np.broadcast_to(idx, (8, 28, 2, 2, 2, 8, 12)).reshape(224, 768)


def _np_band_idx2():
    l = np.arange(8).reshape(8, 1, 1, 1, 1, 1, 1, 1)
    ci = np.arange(8).reshape(1, 8, 1, 1, 1, 1, 1, 1)
    wi = np.arange(12).reshape(1, 1, 12, 1, 1, 1, 1, 1)
    rp = np.arange(2).reshape(1, 1, 1, 2, 1, 1, 1, 1)
    wp = np.arange(2).reshape(1, 1, 1, 1, 2, 1, 1, 1)
    phl = np.arange(2).reshape(1, 1, 1, 1, 1, 2, 1, 1)
    co = np.arange(8).reshape(1, 1, 1, 1, 1, 1, 8, 1)
    pw = np.arange(4).reshape(1, 1, 1, 1, 1, 1, 1, 4)
    kh = (l - 2 * phl - rp) % 8
    kw = (wi - 2 * pw - wp) % 16
    idx = ((co * 8 + ci) * 8 + kh) * 16 + kw
    return np.broadcast_to(idx, (8, 8, 12, 2, 2, 2, 8, 4)).reshape(768, 256)


_IDX1 = _np_band_idx1()
_IDX2 = _np_band_idx2()


def _stage1_body(x_ref, w_ref, b_ref, z_ref, st_ref, *, n_valid, tb):
    xb = x_ref[...].astype(jnp.bfloat16)
    w = w_ref[...]
    chunks = []
    for blk in range(6):
        y = jnp.dot(xb[:, 112 * blk:112 * blk + 224], w,
                    preferred_element_type=jnp.float32)
        p = jnp.maximum(jnp.maximum(y[:, 0:192], y[:, 192:384]),
                        jnp.maximum(y[:, 384:576], y[:, 576:768]))
        chunks.append(p)
    z = jnp.concatenate(chunks, axis=-1)
    z = jnp.maximum(z + b_ref[...], 0.0).astype(z_ref.dtype)
    z_ref[...] = z

    zf = z.astype(jnp.float32)
    if n_valid is not None:
        row = jax.lax.broadcasted_iota(jnp.int32, (tb, 1), 0) + pl.program_id(0) * tb
        zf = jnp.where(row < n_valid, zf, 0.0)
    s = jnp.sum(zf, axis=0, keepdims=True)
    sq = jnp.sum(zf * zf, axis=0, keepdims=True)
    st_ref[...] = jnp.concatenate([s, sq], axis=0)[None]


def _stage2_body(z1_ref, w_ref, b_ref, z_ref, st_ref, *, n_valid, tb):
    zn = z1_ref[...]
    w = w_ref[...]
    chunks = []
    for blk in range(2):
        y = jnp.dot(zn[:, 384 * blk:384 * blk + 768], w,
                    preferred_element_type=jnp.float32)
        p = jnp.maximum(jnp.maximum(y[:, 0:64], y[:, 64:128]),
                        jnp.maximum(y[:, 128:192], y[:, 192:256]))
        chunks.append(p)
    z = jnp.concatenate(chunks, axis=-1)
    z = jnp.maximum(z + b_ref[...], 0.0).astype(z_ref.dtype)
    z_ref[...] = z

    zf = z.astype(jnp.float32)
    if n_valid is not None:
        row = jax.lax.broadcasted_iota(jnp.int32, (tb, 1), 0) + pl.program_id(0) * tb
        zf = jnp.where(row < n_valid, zf, 0.0)
    s = jnp.sum(zf, axis=0, keepdims=True)
    sq = jnp.sum(zf * zf, axis=0, keepdims=True)
    st_ref[...] = jnp.concatenate([s, sq], axis=0)[None]


def _head_body(z2_ref, w1_ref, b1_ref, w2_ref, b2_ref, o_ref):
    h = jnp.dot(z2_ref[...], w1_ref[...], preferred_element_type=jnp.float32)
    h = jnp.maximum(h + b1_ref[...], 0.0)
    logits = jnp.dot(h.astype(jnp.bfloat16), w2_ref[...],
                     preferred_element_type=jnp.float32) + b2_ref[...]
    m = jnp.max(logits, axis=-1, keepdims=True)
    lse = jnp.log(jnp.sum(jnp.exp(logits - m), axis=-1, keepdims=True)) + m
    o_ref[...] = (logits - lse).astype(o_ref.dtype)


def _per_lane(v, w_rep, h_rep):
    return jnp.tile(jnp.repeat(v, w_rep), h_rep)


def _bn_scale_shift(st, n_valid, h_dim, w_dim, gamma, beta, eps=1e-5):
    tot = st.sum(axis=0)
    per_c = tot.reshape(2, h_dim, 8, w_dim).sum(axis=(1, 3))
    count = n_valid * h_dim * w_dim
    mean = per_c[0] / count
    var = per_c[1] / count - mean * mean
    scale = gamma * jax.lax.rsqrt(var + eps)
    shift = beta - mean * scale
    return scale, shift


def _cdiv(a, b):
    return -(-a // b)


def kernel(x, W1, b1, W2, b2, g1, be1, g2, be2, Wf1, bf1, Wf2, bf2):
    n = x.shape[0]
    tb = min(_TB, _cdiv(n, 8) * 8)
    n_pad = tb * _cdiv(n, tb)
    grid = n_pad // tb
    n_valid = None if n_pad == n else n
    cp = pltpu.CompilerParams(dimension_semantics=("parallel",),
                              vmem_limit_bytes=_VMEM)

    x2d = x.reshape(n, 784)
    if n_pad != n:
        x2d = jnp.pad(x2d, ((0, n_pad - n), (0, 0)))

    t1b = jnp.pad(W1[:, 0], ((0, 0), (0, 3), (0, 27))).reshape(-1)
    w1b = t1b[_IDX1].astype(jnp.bfloat16)
    b1v = _per_lane(b1, 12, 12)[None].astype(jnp.float32)
    z1, st1 = pl.pallas_call(
        functools.partial(_stage1_body, n_valid=n_valid, tb=tb),
        grid=(grid,),
        in_specs=[
            pl.BlockSpec((tb, 784), lambda i: (i, 0)),
            pl.BlockSpec((224, 768), lambda i: (0, 0)),
            pl.BlockSpec((1, 1152), lambda i: (0, 0)),
        ],
        out_specs=(
            pl.BlockSpec((tb, 1152), lambda i: (i, 0)),
            pl.BlockSpec((1, 2, 1152), lambda i: (i, 0, 0)),
        ),
        out_shape=(
            jax.ShapeDtypeStruct((n_pad, 1152), jnp.bfloat16),
            jax.ShapeDtypeStruct((grid, 2, 1152), jnp.float32),
        ),
        compiler_params=cp,
    )(x2d, w1b, b1v)

    s1, t1 = _bn_scale_shift(st1, n, 12, 12, g1, be1)

    w2_eff = W2 * s1[None, :, None, None]
    t2b = jnp.pad(w2_eff, ((0, 0), (0, 0), (0, 3), (0, 11))).reshape(-1)
    w2b = t2b[_IDX2].astype(jnp.bfloat16)
    b2_eff = b2 + W2.sum(axis=(2, 3)) @ t1
    b2v = _per_lane(b2_eff, 4, 4)[None].astype(jnp.float32)
    z2, st2 = pl.pallas_call(
        functools.partial(_stage2_body, n_valid=n_valid, tb=tb),
        grid=(grid,),
        in_specs=[
            pl.BlockSpec((tb, 1152), lambda i: (i, 0)),
            pl.BlockSpec((768, 256), lambda i: (0, 0)),
            pl.BlockSpec((1, 128), lambda i: (0, 0)),
        ],
        out_specs=(
            pl.BlockSpec((tb, 128), lambda i: (i, 0)),
            pl.BlockSpec((1, 2, 128), lambda i: (i, 0, 0)),
        ),
        out_shape=(
            jax.ShapeDtypeStruct((n_pad, 128), jnp.bfloat16),
            jax.ShapeDtypeStruct((grid, 2, 128), jnp.float32),
        ),
        compiler_params=cp,
    )(z1, w2b, b2v)

    s2, t2 = _bn_scale_shift(st2, n, 4, 4, g2, be2)

    w1h = Wf1.reshape(64, 8, 4, 4).transpose(0, 2, 1, 3).reshape(64, 128).T
    w1h = w1h * _per_lane(s2, 4, 4)[:, None]
    w1p = jnp.pad(w1h, ((0, 0), (0, 64))).astype(jnp.bfloat16)
    b1h = bf1 + Wf1.reshape(64, 8, 16).sum(axis=-1) @ t2
    b1p = jnp.pad(b1h, (0, 64))[None].astype(jnp.float32)
    w2p = jnp.pad(Wf2.T, ((0, 64), (0, 118))).astype(jnp.bfloat16)
    b2p = jnp.concatenate(
        [bf2.astype(jnp.float32), jnp.full((118,), -1e30, jnp.float32)])[None]
    out = pl.pallas_call(
        _head_body,
        grid=(grid,),
        in_specs=[
            pl.BlockSpec((tb, 128), lambda i: (i, 0)),
            pl.BlockSpec((128, 128), lambda i: (0, 0)),
            pl.BlockSpec((1, 128), lambda i: (0, 0)),
            pl.BlockSpec((128, 128), lambda i: (0, 0)),
            pl.BlockSpec((1, 128), lambda i: (0, 0)),
        ],
        out_specs=pl.BlockSpec((tb, 128), lambda i: (i, 0)),
        out_shape=jax.ShapeDtypeStruct((n_pad, 128), jnp.float32),
        compiler_params=cp,
    )(z2, w1p, b1p, w2p, b2p)
    return out[:n, :10]


# speedup vs baseline: 11.8122x; 11.8122x over previous
import functools

import numpy as np

import jax
import jax.numpy as jnp
from jax.experimental import pallas as pl
from jax.experimental.pallas import tpu as pltpu

_TB = 1024
_VMEM = 100 * 1024 * 1024


def _band_block1(w):
    l = jnp.arange(8); wi = jnp.arange(28)
    phl = jnp.arange(2); rp = jnp.arange(2)
    pw = jnp.arange(12); wp = jnp.arange(2)
    kh = l[:, None, None] - 2 * phl[None, :, None] - rp[None, None, :]
    kw = wi[:, None, None] - 2 * pw[None, :, None] - wp[None, None, :]
    vh = (kh >= 0) & (kh < 5)
    vw = (kw >= 0) & (kw < 5)
    wc = w[:, 0]
    t = wc[:, kh.clip(0, 4), :]
    t = t[..., kw.clip(0, 4)]
    mask = (vh[None, :, :, :, None, None, None]
            & vw[None, None, None, None, :, :, :])
    t = t * mask.astype(w.dtype)
    t = jnp.transpose(t, (1, 4, 3, 6, 2, 0, 5))
    return t.reshape(224, 768)


def _band_block2(w):
    l = jnp.arange(8); wi = jnp.arange(12)
    phl = jnp.arange(2); rp = jnp.arange(2)
    pw = jnp.arange(4); wp = jnp.arange(2)
    kh = l[:, None, None] - 2 * phl[None, :, None] - rp[None, None, :]
    kw = wi[:, None, None] - 2 * pw[None, :, None] - wp[None, None, :]
    vh = (kh >= 0) & (kh < 5)
    vw = (kw >= 0) & (kw < 5)
    t = w[:, :, kh.clip(0, 4), :]
    t = t[..., kw.clip(0, 4)]
    mask = (vh[None, None, :, :, :, None, None, None]
            & vw[None, None, None, None, None, :, :, :])
    t = t * mask.astype(w.dtype)
    t = jnp.transpose(t, (2, 1, 5, 4, 7, 3, 0, 6))
    return t.reshape(768, 256)


def _stage1_body(x_ref, w_ref, b_ref, z_ref, st_ref, *, n_valid, tb):
    xb = x_ref[...].astype(jnp.bfloat16)
    w = w_ref[...]
    chunks = []
    for blk in range(6):
        y = jnp.dot(xb[:, 112 * blk:112 * blk + 224], w,
                    preferred_element_type=jnp.float32)
        p = jnp.maximum(jnp.maximum(y[:, 0:192], y[:, 192:384]),
                        jnp.maximum(y[:, 384:576], y[:, 576:768]))
        chunks.append(p)
    z = jnp.concatenate(chunks, axis=-1)
    z = jnp.maximum(z + b_ref[...], 0.0).astype(z_ref.dtype)
    z_ref[...] = z

    zf = z.astype(jnp.float32)
    if n_valid is not None:
        row = jax.lax.broadcasted_iota(jnp.int32, (tb, 1), 0) + pl.program_id(0) * tb
        zf = jnp.where(row < n_valid, zf, 0.0)
    s = jnp.sum(zf, axis=0, keepdims=True)
    sq = jnp.sum(zf * zf, axis=0, keepdims=True)
    st_ref[...] = jnp.concatenate([s, sq], axis=0)[None]


def _stage2_body(z1_ref, w_ref, b_ref, z_ref, st_ref, *, n_valid, tb):
    zn = z1_ref[...]
    w = w_ref[...]
    chunks = []
    for blk in range(2):
        y = jnp.dot(zn[:, 384 * blk:384 * blk + 768], w,
                    preferred_element_type=jnp.float32)
        p = jnp.maximum(jnp.maximum(y[:, 0:64], y[:, 64:128]),
                        jnp.maximum(y[:, 128:192], y[:, 192:256]))
        chunks.append(p)
    z = jnp.concatenate(chunks, axis=-1)
    z = jnp.maximum(z + b_ref[...], 0.0).astype(z_ref.dtype)
    z_ref[...] = z

    zf = z.astype(jnp.float32)
    if n_valid is not None:
        row = jax.lax.broadcasted_iota(jnp.int32, (tb, 1), 0) + pl.program_id(0) * tb
        zf = jnp.where(row < n_valid, zf, 0.0)
    s = jnp.sum(zf, axis=0, keepdims=True)
    sq = jnp.sum(zf * zf, axis=0, keepdims=True)
    st_ref[...] = jnp.concatenate([s, sq], axis=0)[None]


def _head_body(z2_ref, w1_ref, b1_ref, w2_ref, b2_ref, o_ref):
    h = jnp.dot(z2_ref[...], w1_ref[...], preferred_element_type=jnp.float32)
    h = jnp.maximum(h + b1_ref[...], 0.0)
    logits = jnp.dot(h.astype(jnp.bfloat16), w2_ref[...],
                     preferred_element_type=jnp.float32) + b2_ref[...]
    m = jnp.max(logits, axis=-1, keepdims=True)
    lse = jnp.log(jnp.sum(jnp.exp(logits - m), axis=-1, keepdims=True)) + m
    o_ref[...] = (logits - lse).astype(o_ref.dtype)


def _per_lane(v, w_rep, h_rep):
    return jnp.tile(jnp.repeat(v, w_rep), h_rep)


def _bn_scale_shift(st, n_valid, h_dim, w_dim, gamma, beta, eps=1e-5):
    tot = st.sum(axis=0)
    per_c = tot.reshape(2, h_dim, 8, w_dim).sum(axis=(1, 3))
    count = n_valid * h_dim * w_dim
    mean = per_c[0] / count
    var = per_c[1] / count - mean * mean
    scale = gamma * jax.lax.rsqrt(var + eps)
    shift = beta - mean * scale
    return scale, shift


def _cdiv(a, b):
    return -(-a // b)


def kernel(x, W1, b1, W2, b2, g1, be1, g2, be2, Wf1, bf1, Wf2, bf2):
    n = x.shape[0]
    tb = min(_TB, _cdiv(n, 8) * 8)
    n_pad = tb * _cdiv(n, tb)
    grid = n_pad // tb
    n_valid = None if n_pad == n else n
    cp = pltpu.CompilerParams(dimension_semantics=("parallel",),
                              vmem_limit_bytes=_VMEM)

    x2d = x.reshape(n, 784)
    if n_pad != n:
        x2d = jnp.pad(x2d, ((0, n_pad - n), (0, 0)))

    w1b = _band_block1(W1).astype(jnp.bfloat16)
    b1v = _per_lane(b1, 12, 12)[None].astype(jnp.float32)
    z1, st1 = pl.pallas_call(
        functools.partial(_stage1_body, n_valid=n_valid, tb=tb),
        grid=(grid,),
        in_specs=[
            pl.BlockSpec((tb, 784), lambda i: (i, 0)),
            pl.BlockSpec((224, 768), lambda i: (0, 0)),
            pl.BlockSpec((1, 1152), lambda i: (0, 0)),
        ],
        out_specs=(
            pl.BlockSpec((tb, 1152), lambda i: (i, 0)),
            pl.BlockSpec((1, 2, 1152), lambda i: (i, 0, 0)),
        ),
        out_shape=(
            jax.ShapeDtypeStruct((n_pad, 1152), jnp.bfloat16),
            jax.ShapeDtypeStruct((grid, 2, 1152), jnp.float32),
        ),
        compiler_params=cp,
    )(x2d, w1b, b1v)

    s1, t1 = _bn_scale_shift(st1, n, 12, 12, g1, be1)

    w2_eff = W2 * s1[None, :, None, None]
    w2b = _band_block2(w2_eff).astype(jnp.bfloat16)
    b2_eff = b2 + W2.sum(axis=(2, 3)) @ t1
    b2v = _per_lane(b2_eff, 4, 4)[None].astype(jnp.float32)
    z2, st2 = pl.pallas_call(
        functools.partial(_stage2_body, n_valid=n_valid, tb=tb),
        grid=(grid,),
        in_specs=[
            pl.BlockSpec((tb, 1152), lambda i: (i, 0)),
            pl.BlockSpec((768, 256), lambda i: (0, 0)),
            pl.BlockSpec((1, 128), lambda i: (0, 0)),
        ],
        out_specs=(
            pl.BlockSpec((tb, 128), lambda i: (i, 0)),
            pl.BlockSpec((1, 2, 128), lambda i: (i, 0, 0)),
        ),
        out_shape=(
            jax.ShapeDtypeStruct((n_pad, 128), jnp.bfloat16),
            jax.ShapeDtypeStruct((grid, 2, 128), jnp.float32),
        ),
        compiler_params=cp,
    )(z1, w2b, b2v)

    s2, t2 = _bn_scale_shift(st2, n, 4, 4, g2, be2)

    w1h = Wf1.reshape(64, 8, 4, 4).transpose(0, 2, 1, 3).reshape(64, 128).T
    w1h = w1h * _per_lane(s2, 4, 4)[:, None]
    w1p = jnp.pad(w1h, ((0, 0), (0, 64))).astype(jnp.bfloat16)
    b1h = bf1 + Wf1.reshape(64, 8, 16).sum(axis=-1) @ t2
    b1p = jnp.pad(b1h, (0, 64))[None].astype(jnp.float32)
    w2p = jnp.pad(Wf2.T, ((0, 64), (0, 118))).astype(jnp.bfloat16)
    b2p = jnp.concatenate(
        [bf2.astype(jnp.float32), jnp.full((118,), -1e30, jnp.float32)])[None]
    out = pl.pallas_call(
        _head_body,
        grid=(grid,),
        in_specs=[
            pl.BlockSpec((tb, 128), lambda i: (i, 0)),
            pl.BlockSpec((128, 128), lambda i: (0, 0)),
            pl.BlockSpec((1, 128), lambda i: (0, 0)),
            pl.BlockSpec((128, 128), lambda i: (0, 0)),
            pl.BlockSpec((1, 128), lambda i: (0, 0)),
        ],
        out_specs=pl.BlockSpec((tb, 128), lambda i: (i, 0)),
        out_shape=jax.ShapeDtypeStruct((n_pad, 128), jnp.float32),
        compiler_params=cp,
    )(z2, w1p, b1p, w2p, b2p)
    return out[:n, :10]


# speedup vs baseline: 16.2836x; 1.3785x over previous
import functools

import jax
import jax.numpy as jnp
from jax.experimental import pallas as pl
from jax.experimental.pallas import tpu as pltpu

_TB = 1024
_VMEM = 100 * 1024 * 1024


def _stage1_body(x_ref, w_ref, b_ref, z_ref, st_ref, *, n_valid, tb):
    xb = x_ref[...].astype(jnp.bfloat16)
    w = w_ref[...]
    chunks = []
    for blk in range(6):
        y = jnp.dot(w, xb[112 * blk:112 * blk + 224, :],
                    preferred_element_type=jnp.float32)
        p = jnp.maximum(jnp.maximum(y[0:192], y[192:384]),
                        jnp.maximum(y[384:576], y[576:768]))
        chunks.append(p)
    z = jnp.concatenate(chunks, axis=0)
    z = jnp.maximum(z + b_ref[...], 0.0).astype(z_ref.dtype)
    z_ref[...] = z

    zf = z.astype(jnp.float32)
    if n_valid is not None:
        col = jax.lax.broadcasted_iota(jnp.int32, (1, tb), 1) + pl.program_id(0) * tb
        zf = jnp.where(col < n_valid, zf, 0.0)
    s = jnp.sum(zf, axis=1, keepdims=True)
    sq = jnp.sum(zf * zf, axis=1, keepdims=True)
    st_ref[...] = jnp.concatenate([s, sq], axis=1)[None]


def _stage2_body(z1_ref, w_ref, b_ref, z_ref, st_ref, *, n_valid, tb):
    zn = z1_ref[...]
    w = w_ref[...]
    chunks = []
    for blk in range(2):
        y = jnp.dot(w, zn[384 * blk:384 * blk + 768, :],
                    preferred_element_type=jnp.float32)
        p = jnp.maximum(jnp.maximum(y[0:64], y[64:128]),
                        jnp.maximum(y[128:192], y[192:256]))
        chunks.append(p)
    z = jnp.concatenate(chunks, axis=0)
    z = jnp.maximum(z + b_ref[...], 0.0).astype(z_ref.dtype)
    z_ref[...] = z

    zf = z.astype(jnp.float32)
    if n_valid is not None:
        col = jax.lax.broadcasted_iota(jnp.int32, (1, tb), 1) + pl.program_id(0) * tb
        zf = jnp.where(col < n_valid, zf, 0.0)
    s = jnp.sum(zf, axis=1, keepdims=True)
    sq = jnp.sum(zf * zf, axis=1, keepdims=True)
    st_ref[...] = jnp.concatenate([s, sq], axis=1)[None]


def _head_body(z2_ref, w1_ref, b1_ref, w2_ref, b2_ref, o_ref):
    h = jnp.dot(w1_ref[...], z2_ref[...], preferred_element_type=jnp.float32)
    h = jnp.maximum(h + b1_ref[...], 0.0)
    logits = jnp.dot(w2_ref[...], h.astype(jnp.bfloat16),
                     preferred_element_type=jnp.float32) + b2_ref[...]
    m = jnp.max(logits, axis=0, keepdims=True)
    lse = jnp.log(jnp.sum(jnp.exp(logits - m), axis=0, keepdims=True)) + m
    o_ref[...] = (logits - lse).astype(o_ref.dtype)


def _band_block1_t(w):
    wp_ = jnp.pad(w[:, 0], ((0, 0), (3, 4), (23, 23)))
    groups = []
    for rp in range(2):
        for wq in range(2):
            for phl in range(2):
                s = 2 * phl + rp
                per_pw = [
                    jax.lax.slice(wp_, (0, 3 - s, 23 - (2 * pw + wq)),
                                  (8, 11 - s, 51 - (2 * pw + wq)))
                    for pw in range(12)
                ]
                groups.append(jnp.stack(per_pw, axis=1))
    return jnp.stack(groups, axis=0).reshape(768, 224)


def _band_block2_t(w):
    wt = jnp.transpose(w, (0, 2, 1, 3))
    wp_ = jnp.pad(wt, ((0, 0), (3, 4), (0, 0), (7, 7)))
    groups = []
    for rp in range(2):
        for wq in range(2):
            for phl in range(2):
                s = 2 * phl + rp
                per_pw = [
                    jax.lax.slice(wp_, (0, 3 - s, 0, 7 - (2 * pw + wq)),
                                  (8, 11 - s, 8, 19 - (2 * pw + wq)))
                    for pw in range(4)
                ]
                groups.append(jnp.stack(per_pw, axis=1))
    return jnp.stack(groups, axis=0).reshape(256, 768)


def _per_row(v, w_rep, h_rep):
    return jnp.tile(jnp.repeat(v, w_rep), h_rep)


def _bn_scale_shift(st, n_valid, h_dim, w_dim, gamma, beta, eps=1e-5):
    tot = st.sum(axis=0)
    per_c = tot.reshape(h_dim, 8, w_dim, 2).sum(axis=(0, 2))
    count = n_valid * h_dim * w_dim
    mean = per_c[:, 0] / count
    var = per_c[:, 1] / count - mean * mean
    scale = gamma * jax.lax.rsqrt(var + eps)
    shift = beta - mean * scale
    return scale, shift


def _cdiv(a, b):
    return -(-a // b)


def kernel(x, W1, b1, W2, b2, g1, be1, g2, be2, Wf1, bf1, Wf2, bf2):
    n = x.shape[0]
    tb = min(_TB, _cdiv(n, 128) * 128)
    n_pad = tb * _cdiv(n, tb)
    grid = n_pad // tb
    n_valid = None if n_pad == n else n
    cp = pltpu.CompilerParams(dimension_semantics=("parallel",),
                              vmem_limit_bytes=_VMEM)

    xt = x.reshape(n, 784).T
    if n_pad != n:
        xt = jnp.pad(xt, ((0, 0), (0, n_pad - n)))

    w1b = _band_block1_t(W1).astype(jnp.bfloat16)
    b1v = _per_row(b1, 12, 12)[:, None].astype(jnp.float32)
    z1, st1 = pl.pallas_call(
        functools.partial(_stage1_body, n_valid=n_valid, tb=tb),
        grid=(grid,),
        in_specs=[
            pl.BlockSpec((784, tb), lambda i: (0, i)),
            pl.BlockSpec((768, 224), lambda i: (0, 0)),
            pl.BlockSpec((1152, 1), lambda i: (0, 0)),
        ],
        out_specs=(
            pl.BlockSpec((1152, tb), lambda i: (0, i)),
            pl.BlockSpec((1, 1152, 2), lambda i: (i, 0, 0)),
        ),
        out_shape=(
            jax.ShapeDtypeStruct((1152, n_pad), jnp.bfloat16),
            jax.ShapeDtypeStruct((grid, 1152, 2), jnp.float32),
        ),
        compiler_params=cp,
    )(xt, w1b, b1v)

    s1, t1 = _bn_scale_shift(st1, n, 12, 12, g1, be1)

    w2_eff = W2 * s1[None, :, None, None]
    w2b = _band_block2_t(w2_eff).astype(jnp.bfloat16)
    b2_eff = b2 + W2.sum(axis=(2, 3)) @ t1
    b2v = _per_row(b2_eff, 4, 4)[:, None].astype(jnp.float32)
    z2, st2 = pl.pallas_call(
        functools.partial(_stage2_body, n_valid=n_valid, tb=tb),
        grid=(grid,),
        in_specs=[
            pl.BlockSpec((1152, tb), lambda i: (0, i)),
            pl.BlockSpec((256, 768), lambda i: (0, 0)),
            pl.BlockSpec((128, 1), lambda i: (0, 0)),
        ],
        out_specs=(
            pl.BlockSpec((128, tb), lambda i: (0, i)),
            pl.BlockSpec((1, 128, 2), lambda i: (i, 0, 0)),
        ),
        out_shape=(
            jax.ShapeDtypeStruct((128, n_pad), jnp.bfloat16),
            jax.ShapeDtypeStruct((grid, 128, 2), jnp.float32),
        ),
        compiler_params=cp,
    )(z1, w2b, b2v)

    s2, t2 = _bn_scale_shift(st2, n, 4, 4, g2, be2)

    w1h = Wf1.reshape(64, 8, 4, 4).transpose(0, 2, 1, 3).reshape(64, 128)
    w1h = w1h * _per_row(s2, 4, 4)[None, :]
    w1p = jnp.pad(w1h, ((0, 64), (0, 0))).astype(jnp.bfloat16)
    b1h = bf1 + Wf1.reshape(64, 8, 16).sum(axis=-1) @ t2
    b1p = jnp.pad(b1h, (0, 64))[:, None].astype(jnp.float32)
    w2p = jnp.pad(Wf2, ((0, 118), (0, 64))).astype(jnp.bfloat16)
    b2p = jnp.concatenate(
        [bf2.astype(jnp.float32), jnp.full((118,), -1e30, jnp.float32)])[:, None]
    out = pl.pallas_call(
        _head_body,
        grid=(grid,),
        in_specs=[
            pl.BlockSpec((128, tb), lambda i: (0, i)),
            pl.BlockSpec((128, 128), lambda i: (0, 0)),
            pl.BlockSpec((128, 1), lambda i: (0, 0)),
            pl.BlockSpec((128, 128), lambda i: (0, 0)),
            pl.BlockSpec((128, 1), lambda i: (0, 0)),
        ],
        out_specs=pl.BlockSpec((128, tb), lambda i: (0, i)),
        out_shape=jax.ShapeDtypeStruct((128, n_pad), jnp.float32),
        compiler_params=cp,
    )(z2, w1p, b1p, w2p, b2p)
    return out[:10, :n].T


# speedup vs baseline: 18.5556x; 1.1395x over previous
import functools

import jax
import jax.numpy as jnp
from jax.experimental import pallas as pl
from jax.experimental.pallas import tpu as pltpu

_TB = 1024
_VMEM = 100 * 1024 * 1024


def _stage1_body(x_ref, w_ref, b_ref, z_ref, st_ref, *, n_valid, tb):
    xb = x_ref[...]
    w = w_ref[...]
    chunks = []
    for blk in range(6):
        y = jnp.dot(w, xb[112 * blk:112 * blk + 224, :],
                    preferred_element_type=jnp.float32)
        p = jnp.maximum(jnp.maximum(y[0:192], y[192:384]),
                        jnp.maximum(y[384:576], y[576:768]))
        chunks.append(p)
    z = jnp.concatenate(chunks, axis=0)
    z = jnp.maximum(z + b_ref[...], 0.0).astype(z_ref.dtype)
    z_ref[...] = z

    zf = z.astype(jnp.float32)
    if n_valid is not None:
        col = jax.lax.broadcasted_iota(jnp.int32, (1, tb), 1) + pl.program_id(0) * tb
        zf = jnp.where(col < n_valid, zf, 0.0)
    s = jnp.sum(zf, axis=1, keepdims=True)
    sq = jnp.sum(zf * zf, axis=1, keepdims=True)
    st_ref[...] = jnp.concatenate([s, sq], axis=1)[None]


def _stage2_body(z1_ref, w_ref, b_ref, z_ref, st_ref, *, n_valid, tb):
    zn = z1_ref[...]
    w = w_ref[...]
    chunks = []
    for blk in range(2):
        y = jnp.dot(w, zn[384 * blk:384 * blk + 768, :],
                    preferred_element_type=jnp.float32)
        p = jnp.maximum(jnp.maximum(y[0:64], y[64:128]),
                        jnp.maximum(y[128:192], y[192:256]))
        chunks.append(p)
    z = jnp.concatenate(chunks, axis=0)
    z = jnp.maximum(z + b_ref[...], 0.0).astype(z_ref.dtype)
    z_ref[...] = z

    zf = z.astype(jnp.float32)
    if n_valid is not None:
        col = jax.lax.broadcasted_iota(jnp.int32, (1, tb), 1) + pl.program_id(0) * tb
        zf = jnp.where(col < n_valid, zf, 0.0)
    s = jnp.sum(zf, axis=1, keepdims=True)
    sq = jnp.sum(zf * zf, axis=1, keepdims=True)
    st_ref[...] = jnp.concatenate([s, sq], axis=1)[None]


def _head_body(z2_ref, w1_ref, b1_ref, w2_ref, b2_ref, o_ref):
    h = jnp.dot(w1_ref[...], z2_ref[...], preferred_element_type=jnp.float32)
    h = jnp.maximum(h + b1_ref[...], 0.0)
    logits = jnp.dot(w2_ref[...], h.astype(jnp.bfloat16),
                     preferred_element_type=jnp.float32) + b2_ref[...]
    m = jnp.max(logits, axis=0, keepdims=True)
    lse = jnp.log(jnp.sum(jnp.exp(logits - m), axis=0, keepdims=True)) + m
    o_ref[...] = (logits - lse)[:16, :].astype(o_ref.dtype)


def _band_block1_t(w):
    wp_ = jnp.pad(w[:, 0], ((0, 0), (3, 4), (23, 23)))
    groups = []
    for rp in range(2):
        for wq in range(2):
            for phl in range(2):
                s = 2 * phl + rp
                per_pw = [
                    jax.lax.slice(wp_, (0, 3 - s, 23 - (2 * pw + wq)),
                                  (8, 11 - s, 51 - (2 * pw + wq)))
                    for pw in range(12)
                ]
                groups.append(jnp.stack(per_pw, axis=1))
    return jnp.stack(groups, axis=0).reshape(768, 224)


def _band_block2_t(w):
    wt = jnp.transpose(w, (0, 2, 1, 3))
    wp_ = jnp.pad(wt, ((0, 0), (3, 4), (0, 0), (7, 7)))
    groups = []
    for rp in range(2):
        for wq in range(2):
            for phl in range(2):
                s = 2 * phl + rp
                per_pw = [
                    jax.lax.slice(wp_, (0, 3 - s, 0, 7 - (2 * pw + wq)),
                                  (8, 11 - s, 8, 19 - (2 * pw + wq)))
                    for pw in range(4)
                ]
                groups.append(jnp.stack(per_pw, axis=1))
    return jnp.stack(groups, axis=0).reshape(256, 768)


def _per_row(v, w_rep, h_rep):
    return jnp.tile(jnp.repeat(v, w_rep), h_rep)


def _bn_scale_shift(st, n_valid, h_dim, w_dim, gamma, beta, eps=1e-5):
    tot = st.sum(axis=0)
    per_c = tot.reshape(h_dim, 8, w_dim, 2).sum(axis=(0, 2))
    count = n_valid * h_dim * w_dim
    mean = per_c[:, 0] / count
    var = per_c[:, 1] / count - mean * mean
    scale = gamma * jax.lax.rsqrt(var + eps)
    shift = beta - mean * scale
    return scale, shift


def _cdiv(a, b):
    return -(-a // b)


def kernel(x, W1, b1, W2, b2, g1, be1, g2, be2, Wf1, bf1, Wf2, bf2):
    n = x.shape[0]
    tb = min(_TB, _cdiv(n, 128) * 128)
    n_pad = tb * _cdiv(n, tb)
    grid = n_pad // tb
    n_valid = None if n_pad == n else n
    cp = pltpu.CompilerParams(dimension_semantics=("parallel",),
                              vmem_limit_bytes=_VMEM)

    xt = jnp.transpose(x[:, 0], (1, 2, 0)).reshape(784, n).astype(jnp.bfloat16)
    if n_pad != n:
        xt = jnp.pad(xt, ((0, 0), (0, n_pad - n)))

    w1b = _band_block1_t(W1).astype(jnp.bfloat16)
    b1v = _per_row(b1, 12, 12)[:, None].astype(jnp.float32)
    z1, st1 = pl.pallas_call(
        functools.partial(_stage1_body, n_valid=n_valid, tb=tb),
        grid=(grid,),
        in_specs=[
            pl.BlockSpec((784, tb), lambda i: (0, i)),
            pl.BlockSpec((768, 224), lambda i: (0, 0)),
            pl.BlockSpec((1152, 1), lambda i: (0, 0)),
        ],
        out_specs=(
            pl.BlockSpec((1152, tb), lambda i: (0, i)),
            pl.BlockSpec((1, 1152, 2), lambda i: (i, 0, 0)),
        ),
        out_shape=(
            jax.ShapeDtypeStruct((1152, n_pad), jnp.bfloat16),
            jax.ShapeDtypeStruct((grid, 1152, 2), jnp.float32),
        ),
        compiler_params=cp,
    )(xt, w1b, b1v)

    s1, t1 = _bn_scale_shift(st1, n, 12, 12, g1, be1)

    w2_eff = W2 * s1[None, :, None, None]
    w2b = _band_block2_t(w2_eff).astype(jnp.bfloat16)
    b2_eff = b2 + W2.sum(axis=(2, 3)) @ t1
    b2v = _per_row(b2_eff, 4, 4)[:, None].astype(jnp.float32)
    z2, st2 = pl.pallas_call(
        functools.partial(_stage2_body, n_valid=n_valid, tb=tb),
        grid=(grid,),
        in_specs=[
            pl.BlockSpec((1152, tb), lambda i: (0, i)),
            pl.BlockSpec((256, 768), lambda i: (0, 0)),
            pl.BlockSpec((128, 1), lambda i: (0, 0)),
        ],
        out_specs=(
            pl.BlockSpec((128, tb), lambda i: (0, i)),
            pl.BlockSpec((1, 128, 2), lambda i: (i, 0, 0)),
        ),
        out_shape=(
            jax.ShapeDtypeStruct((128, n_pad), jnp.bfloat16),
            jax.ShapeDtypeStruct((grid, 128, 2), jnp.float32),
        ),
        compiler_params=cp,
    )(z1, w2b, b2v)

    s2, t2 = _bn_scale_shift(st2, n, 4, 4, g2, be2)

    w1h = Wf1.reshape(64, 8, 4, 4).transpose(0, 2, 1, 3).reshape(64, 128)
    w1h = w1h * _per_row(s2, 4, 4)[None, :]
    w1p = jnp.pad(w1h, ((0, 64), (0, 0))).astype(jnp.bfloat16)
    b1h = bf1 + Wf1.reshape(64, 8, 16).sum(axis=-1) @ t2
    b1p = jnp.pad(b1h, (0, 64))[:, None].astype(jnp.float32)
    w2p = jnp.pad(Wf2, ((0, 118), (0, 64))).astype(jnp.bfloat16)
    b2p = jnp.concatenate(
        [bf2.astype(jnp.float32), jnp.full((118,), -1e30, jnp.float32)])[:, None]
    out = pl.pallas_call(
        _head_body,
        grid=(grid,),
        in_specs=[
            pl.BlockSpec((128, tb), lambda i: (0, i)),
            pl.BlockSpec((128, 128), lambda i: (0, 0)),
            pl.BlockSpec((128, 1), lambda i: (0, 0)),
            pl.BlockSpec((128, 128), lambda i: (0, 0)),
            pl.BlockSpec((128, 1), lambda i: (0, 0)),
        ],
        out_specs=pl.BlockSpec((16, tb), lambda i: (0, i)),
        out_shape=jax.ShapeDtypeStruct((16, n_pad), jnp.float32),
        compiler_params=cp,
    )(z2, w1p, b1p, w2p, b2p)
    return out[:10, :n].T


# speedup vs baseline: 20.8668x; 1.1246x over previous
import functools

import jax
import jax.numpy as jnp
from jax.experimental import pallas as pl
from jax.experimental.pallas import tpu as pltpu

_TB = 1024
_VMEM = 100 * 1024 * 1024


def _stage1_body(x_ref, w_ref, b_ref, z_ref, st_ref, *, n_valid, tb):
    xb = x_ref[...].astype(jnp.bfloat16)
    w = w_ref[...]
    chunks = []
    for blk in range(6):
        y = jnp.dot(w, xb[112 * blk:112 * blk + 224, :],
                    preferred_element_type=jnp.float32)
        p = jnp.maximum(jnp.maximum(y[0:192], y[192:384]),
                        jnp.maximum(y[384:576], y[576:768]))
        chunks.append(p)
    z = jnp.concatenate(chunks, axis=0)
    z = jnp.maximum(z + b_ref[...], 0.0).astype(z_ref.dtype)
    z_ref[...] = z

    zf = z.astype(jnp.float32)
    if n_valid is not None:
        col = jax.lax.broadcasted_iota(jnp.int32, (1, tb), 1) + pl.program_id(0) * tb
        zf = jnp.where(col < n_valid, zf, 0.0)
    s = jnp.sum(zf, axis=1, keepdims=True)
    sq = jnp.sum(zf * zf, axis=1, keepdims=True)
    st_ref[...] = jnp.concatenate([s, sq], axis=1)[None]


def _stage2_body(z1_ref, w_ref, b_ref, z_ref, st_ref, *, n_valid, tb):
    zn = z1_ref[...]
    w = w_ref[...]
    chunks = []
    for blk in range(2):
        y = jnp.dot(w, zn[384 * blk:384 * blk + 768, :],
                    preferred_element_type=jnp.float32)
        p = jnp.maximum(jnp.maximum(y[0:64], y[64:128]),
                        jnp.maximum(y[128:192], y[192:256]))
        chunks.append(p)
    z = jnp.concatenate(chunks, axis=0)
    z = jnp.maximum(z + b_ref[...], 0.0).astype(z_ref.dtype)
    z_ref[...] = z

    zf = z.astype(jnp.float32)
    if n_valid is not None:
        col = jax.lax.broadcasted_iota(jnp.int32, (1, tb), 1) + pl.program_id(0) * tb
        zf = jnp.where(col < n_valid, zf, 0.0)
    s = jnp.sum(zf, axis=1, keepdims=True)
    sq = jnp.sum(zf * zf, axis=1, keepdims=True)
    st_ref[...] = jnp.concatenate([s, sq], axis=1)[None]


def _head_body(z2_ref, w1_ref, b1_ref, w2_ref, b2_ref, o_ref):
    h = jnp.dot(w1_ref[...], z2_ref[...], preferred_element_type=jnp.float32)
    h = jnp.maximum(h + b1_ref[...], 0.0)
    logits = jnp.dot(w2_ref[...], h.astype(jnp.bfloat16),
                     preferred_element_type=jnp.float32) + b2_ref[...]
    m = jnp.max(logits, axis=0, keepdims=True)
    lse = jnp.log(jnp.sum(jnp.exp(logits - m), axis=0, keepdims=True)) + m
    o_ref[...] = (logits - lse)[:16, :].astype(o_ref.dtype)


def _band_block1_t(w):
    wp_ = jnp.pad(w[:, 0], ((0, 0), (3, 4), (23, 23)))
    groups = []
    for rp in range(2):
        for wq in range(2):
            for phl in range(2):
                s = 2 * phl + rp
                per_pw = [
                    jax.lax.slice(wp_, (0, 3 - s, 23 - (2 * pw + wq)),
                                  (8, 11 - s, 51 - (2 * pw + wq)))
                    for pw in range(12)
                ]
                groups.append(jnp.stack(per_pw, axis=1))
    return jnp.stack(groups, axis=0).reshape(768, 224)


def _band_block2_t(w):
    wt = jnp.transpose(w, (0, 2, 1, 3))
    wp_ = jnp.pad(wt, ((0, 0), (3, 4), (0, 0), (7, 7)))
    groups = []
    for rp in range(2):
        for wq in range(2):
            for phl in range(2):
                s = 2 * phl + rp
                per_pw = [
                    jax.lax.slice(wp_, (0, 3 - s, 0, 7 - (2 * pw + wq)),
                                  (8, 11 - s, 8, 19 - (2 * pw + wq)))
                    for pw in range(4)
                ]
                groups.append(jnp.stack(per_pw, axis=1))
    return jnp.stack(groups, axis=0).reshape(256, 768)


def _per_row(v, w_rep, h_rep):
    return jnp.tile(jnp.repeat(v, w_rep), h_rep)


def _bn_scale_shift(st, n_valid, h_dim, w_dim, gamma, beta, eps=1e-5):
    tot = st.sum(axis=0)
    per_c = tot.reshape(h_dim, 8, w_dim, 2).sum(axis=(0, 2))
    count = n_valid * h_dim * w_dim
    mean = per_c[:, 0] / count
    var = per_c[:, 1] / count - mean * mean
    scale = gamma * jax.lax.rsqrt(var + eps)
    shift = beta - mean * scale
    return scale, shift


def _cdiv(a, b):
    return -(-a // b)


def kernel(x, W1, b1, W2, b2, g1, be1, g2, be2, Wf1, bf1, Wf2, bf2):
    n = x.shape[0]
    tb = min(_TB, _cdiv(n, 128) * 128)
    n_pad = tb * _cdiv(n, tb)
    grid = n_pad // tb
    n_valid = None if n_pad == n else n
    cp = pltpu.CompilerParams(dimension_semantics=("parallel",),
                              vmem_limit_bytes=_VMEM)

    xt = jnp.transpose(x[:, 0], (1, 2, 0)).reshape(784, n)
    if n_pad != n:
        xt = jnp.pad(xt, ((0, 0), (0, n_pad - n)))

    w1b = _band_block1_t(W1).astype(jnp.bfloat16)
    b1v = _per_row(b1, 12, 12)[:, None].astype(jnp.float32)
    z1, st1 = pl.pallas_call(
        functools.partial(_stage1_body, n_valid=n_valid, tb=tb),
        grid=(grid,),
        in_specs=[
            pl.BlockSpec((784, tb), lambda i: (0, i)),
            pl.BlockSpec((768, 224), lambda i: (0, 0)),
            pl.BlockSpec((1152, 1), lambda i: (0, 0)),
        ],
        out_specs=(
            pl.BlockSpec((1152, tb), lambda i: (0, i)),
            pl.BlockSpec((1, 1152, 2), lambda i: (i, 0, 0)),
        ),
        out_shape=(
            jax.ShapeDtypeStruct((1152, n_pad), jnp.bfloat16),
            jax.ShapeDtypeStruct((grid, 1152, 2), jnp.float32),
        ),
        compiler_params=cp,
    )(xt, w1b, b1v)

    s1, t1 = _bn_scale_shift(st1, n, 12, 12, g1, be1)

    w2_eff = W2 * s1[None, :, None, None]
    w2b = _band_block2_t(w2_eff).astype(jnp.bfloat16)
    b2_eff = b2 + W2.sum(axis=(2, 3)) @ t1
    b2v = _per_row(b2_eff, 4, 4)[:, None].astype(jnp.float32)
    z2, st2 = pl.pallas_call(
        functools.partial(_stage2_body, n_valid=n_valid, tb=tb),
        grid=(grid,),
        in_specs=[
            pl.BlockSpec((1152, tb), lambda i: (0, i)),
            pl.BlockSpec((256, 768), lambda i: (0, 0)),
            pl.BlockSpec((128, 1), lambda i: (0, 0)),
        ],
        out_specs=(
            pl.BlockSpec((128, tb), lambda i: (0, i)),
            pl.BlockSpec((1, 128, 2), lambda i: (i, 0, 0)),
        ),
        out_shape=(
            jax.ShapeDtypeStruct((128, n_pad), jnp.bfloat16),
            jax.ShapeDtypeStruct((grid, 128, 2), jnp.float32),
        ),
        compiler_params=cp,
    )(z1, w2b, b2v)

    s2, t2 = _bn_scale_shift(st2, n, 4, 4, g2, be2)

    w1h = Wf1.reshape(64, 8, 4, 4).transpose(0, 2, 1, 3).reshape(64, 128)
    w1h = w1h * _per_row(s2, 4, 4)[None, :]
    w1p = jnp.pad(w1h, ((0, 64), (0, 0))).astype(jnp.bfloat16)
    b1h = bf1 + Wf1.reshape(64, 8, 16).sum(axis=-1) @ t2
    b1p = jnp.pad(b1h, (0, 64))[:, None].astype(jnp.float32)
    w2p = jnp.pad(Wf2, ((0, 118), (0, 64))).astype(jnp.bfloat16)
    b2p = jnp.concatenate(
        [bf2.astype(jnp.float32), jnp.full((118,), -1e30, jnp.float32)])[:, None]
    out = pl.pallas_call(
        _head_body,
        grid=(grid,),
        in_specs=[
            pl.BlockSpec((128, tb), lambda i: (0, i)),
            pl.BlockSpec((128, 128), lambda i: (0, 0)),
            pl.BlockSpec((128, 1), lambda i: (0, 0)),
            pl.BlockSpec((128, 128), lambda i: (0, 0)),
            pl.BlockSpec((128, 1), lambda i: (0, 0)),
        ],
        out_specs=pl.BlockSpec((16, tb), lambda i: (0, i)),
        out_shape=jax.ShapeDtypeStruct((16, n_pad), jnp.float32),
        compiler_params=cp,
    )(z2, w1p, b1p, w2p, b2p)
    return out[:10, :n].T


# speedup vs baseline: 21.5728x; 1.0338x over previous
import functools

import jax
import jax.numpy as jnp
from jax.experimental import pallas as pl
from jax.experimental.pallas import tpu as pltpu

_TB = 2048
_VMEM = 100 * 1024 * 1024


def _stage1_body(x_ref, w_ref, b_ref, z_ref, st_ref, *, n_valid, tb):
    xb = x_ref[...].astype(jnp.bfloat16)
    w = w_ref[...]
    chunks = []
    for blk in range(6):
        y = jnp.dot(w, xb[112 * blk:112 * blk + 224, :],
                    preferred_element_type=jnp.float32)
        p = jnp.maximum(jnp.maximum(y[0:192], y[192:384]),
                        jnp.maximum(y[384:576], y[576:768]))
        chunks.append(p)
    z = jnp.concatenate(chunks, axis=0)
    z = jnp.maximum(z + b_ref[...], 0.0).astype(z_ref.dtype)
    z_ref[...] = z

    zf = z.astype(jnp.float32)
    if n_valid is not None:
        col = jax.lax.broadcasted_iota(jnp.int32, (1, tb), 1) + pl.program_id(0) * tb
        zf = jnp.where(col < n_valid, zf, 0.0)
    s = jnp.sum(zf, axis=1, keepdims=True)
    sq = jnp.sum(zf * zf, axis=1, keepdims=True)
    st_ref[...] = jnp.concatenate([s, sq], axis=1)[None]


def _stage2_body(z1_ref, w_ref, b_ref, z_ref, st_ref, *, n_valid, tb):
    zn = z1_ref[...]
    w = w_ref[...]
    chunks = []
    for blk in range(2):
        y = jnp.dot(w, zn[384 * blk:384 * blk + 768, :],
                    preferred_element_type=jnp.float32)
        p = jnp.maximum(jnp.maximum(y[0:64], y[64:128]),
                        jnp.maximum(y[128:192], y[192:256]))
        chunks.append(p)
    z = jnp.concatenate(chunks, axis=0)
    z = jnp.maximum(z + b_ref[...], 0.0).astype(z_ref.dtype)
    z_ref[...] = z

    zf = z.astype(jnp.float32)
    if n_valid is not None:
        col = jax.lax.broadcasted_iota(jnp.int32, (1, tb), 1) + pl.program_id(0) * tb
        zf = jnp.where(col < n_valid, zf, 0.0)
    s = jnp.sum(zf, axis=1, keepdims=True)
    sq = jnp.sum(zf * zf, axis=1, keepdims=True)
    st_ref[...] = jnp.concatenate([s, sq], axis=1)[None]


def _head_body(z2_ref, w1_ref, b1_ref, w2_ref, b2_ref, o_ref):
    h = jnp.dot(w1_ref[...], z2_ref[...], preferred_element_type=jnp.float32)
    h = jnp.maximum(h + b1_ref[...], 0.0)
    logits = jnp.dot(w2_ref[...], h.astype(jnp.bfloat16),
                     preferred_element_type=jnp.float32) + b2_ref[...]
    m = jnp.max(logits, axis=0, keepdims=True)
    lse = jnp.log(jnp.sum(jnp.exp(logits - m), axis=0, keepdims=True)) + m
    o_ref[...] = (logits - lse)[:16, :].astype(o_ref.dtype)


def _band_block1_t(w):
    wp_ = jnp.pad(w[:, 0], ((0, 0), (3, 4), (23, 23)))
    groups = []
    for rp in range(2):
        for wq in range(2):
            for phl in range(2):
                s = 2 * phl + rp
                per_pw = [
                    jax.lax.slice(wp_, (0, 3 - s, 23 - (2 * pw + wq)),
                                  (8, 11 - s, 51 - (2 * pw + wq)))
                    for pw in range(12)
                ]
                groups.append(jnp.stack(per_pw, axis=1))
    return jnp.stack(groups, axis=0).reshape(768, 224)


def _band_block2_t(w):
    wt = jnp.transpose(w, (0, 2, 1, 3))
    wp_ = jnp.pad(wt, ((0, 0), (3, 4), (0, 0), (7, 7)))
    groups = []
    for rp in range(2):
        for wq in range(2):
            for phl in range(2):
                s = 2 * phl + rp
                per_pw = [
                    jax.lax.slice(wp_, (0, 3 - s, 0, 7 - (2 * pw + wq)),
                                  (8, 11 - s, 8, 19 - (2 * pw + wq)))
                    for pw in range(4)
                ]
                groups.append(jnp.stack(per_pw, axis=1))
    return jnp.stack(groups, axis=0).reshape(256, 768)


def _per_row(v, w_rep, h_rep):
    return jnp.tile(jnp.repeat(v, w_rep), h_rep)


def _bn_scale_shift(st, n_valid, h_dim, w_dim, gamma, beta, eps=1e-5):
    tot = st.sum(axis=0)
    per_c = tot.reshape(h_dim, 8, w_dim, 2).sum(axis=(0, 2))
    count = n_valid * h_dim * w_dim
    mean = per_c[:, 0] / count
    var = per_c[:, 1] / count - mean * mean
    scale = gamma * jax.lax.rsqrt(var + eps)
    shift = beta - mean * scale
    return scale, shift


def _cdiv(a, b):
    return -(-a // b)


def kernel(x, W1, b1, W2, b2, g1, be1, g2, be2, Wf1, bf1, Wf2, bf2):
    n = x.shape[0]
    tb = min(_TB, _cdiv(n, 128) * 128)
    n_pad = tb * _cdiv(n, tb)
    grid = n_pad // tb
    n_valid = None if n_pad == n else n
    cp = pltpu.CompilerParams(dimension_semantics=("parallel",),
                              vmem_limit_bytes=_VMEM)

    xt = jnp.transpose(x[:, 0], (1, 2, 0)).reshape(784, n)
    if n_pad != n:
        xt = jnp.pad(xt, ((0, 0), (0, n_pad - n)))

    w1b = _band_block1_t(W1).astype(jnp.bfloat16)
    b1v = _per_row(b1, 12, 12)[:, None].astype(jnp.float32)
    z1, st1 = pl.pallas_call(
        functools.partial(_stage1_body, n_valid=n_valid, tb=tb),
        grid=(grid,),
        in_specs=[
            pl.BlockSpec((784, tb), lambda i: (0, i)),
            pl.BlockSpec((768, 224), lambda i: (0, 0)),
            pl.BlockSpec((1152, 1), lambda i: (0, 0)),
        ],
        out_specs=(
            pl.BlockSpec((1152, tb), lambda i: (0, i)),
            pl.BlockSpec((1, 1152, 2), lambda i: (i, 0, 0)),
        ),
        out_shape=(
            jax.ShapeDtypeStruct((1152, n_pad), jnp.bfloat16),
            jax.ShapeDtypeStruct((grid, 1152, 2), jnp.float32),
        ),
        compiler_params=cp,
    )(xt, w1b, b1v)

    s1, t1 = _bn_scale_shift(st1, n, 12, 12, g1, be1)

    w2_eff = W2 * s1[None, :, None, None]
    w2b = _band_block2_t(w2_eff).astype(jnp.bfloat16)
    b2_eff = b2 + W2.sum(axis=(2, 3)) @ t1
    b2v = _per_row(b2_eff, 4, 4)[:, None].astype(jnp.float32)
    z2, st2 = pl.pallas_call(
        functools.partial(_stage2_body, n_valid=n_valid, tb=tb),
        grid=(grid,),
        in_specs=[
            pl.BlockSpec((1152, tb), lambda i: (0, i)),
            pl.BlockSpec((256, 768), lambda i: (0, 0)),
            pl.BlockSpec((128, 1), lambda i: (0, 0)),
        ],
        out_specs=(
            pl.BlockSpec((128, tb), lambda i: (0, i)),
            pl.BlockSpec((1, 128, 2), lambda i: (i, 0, 0)),
        ),
        out_shape=(
            jax.ShapeDtypeStruct((128, n_pad), jnp.bfloat16),
            jax.ShapeDtypeStruct((grid, 128, 2), jnp.float32),
        ),
        compiler_params=cp,
    )(z1, w2b, b2v)

    s2, t2 = _bn_scale_shift(st2, n, 4, 4, g2, be2)

    w1h = Wf1.reshape(64, 8, 4, 4).transpose(0, 2, 1, 3).reshape(64, 128)
    w1h = w1h * _per_row(s2, 4, 4)[None, :]
    w1p = jnp.pad(w1h, ((0, 64), (0, 0))).astype(jnp.bfloat16)
    b1h = bf1 + Wf1.reshape(64, 8, 16).sum(axis=-1) @ t2
    b1p = jnp.pad(b1h, (0, 64))[:, None].astype(jnp.float32)
    w2p = jnp.pad(Wf2, ((0, 118), (0, 64))).astype(jnp.bfloat16)
    b2p = jnp.concatenate(
        [bf2.astype(jnp.float32), jnp.full((118,), -1e30, jnp.float32)])[:, None]
    out = pl.pallas_call(
        _head_body,
        grid=(grid,),
        in_specs=[
            pl.BlockSpec((128, tb), lambda i: (0, i)),
            pl.BlockSpec((128, 128), lambda i: (0, 0)),
            pl.BlockSpec((128, 1), lambda i: (0, 0)),
            pl.BlockSpec((128, 128), lambda i: (0, 0)),
            pl.BlockSpec((128, 1), lambda i: (0, 0)),
        ],
        out_specs=pl.BlockSpec((16, tb), lambda i: (0, i)),
        out_shape=jax.ShapeDtypeStruct((16, n_pad), jnp.float32),
        compiler_params=cp,
    )(z2, w1p, b1p, w2p, b2p)
    return out[:10, :n].T
